# baseline (speedup 1.0000x reference)
import functools

import jax
import jax.numpy as jnp
import numpy as np
from jax import lax
from jax.experimental import pallas as pl
from jax.experimental.pallas import tpu as pltpu
from jax.experimental.pallas import tpu_sc as plsc

N_NODES = 10000
N_EDGES = 320000
D_IN = 128
HID = 64
NODES_PER_GRAPH = 100
N_GRAPHS = N_NODES // NODES_PER_GRAPH
N_OUT = 10

NC = 2
NS = 16
NW = NC * NS

CHUNK = 128
N_ROWS = 2560
E_PAD = N_ROWS * CHUNK
ROWS_PER_WORKER = N_ROWS // NW
N_PAD = 12288
SLICE_PER_TILE = N_PAD // NS
DIS_PER_WORKER = N_PAD // NW
N_TRIM = 10048

_mesh = plsc.VectorSubcoreMesh(core_axis_name="c", subcore_axis_name="s")
_sc_params = pltpu.CompilerParams(use_tc_tiling_on_sc=False,
                                  needs_layout_passes=False)


def _rsqrt16(d):
    i = lax.bitcast_convert_type(d, jnp.int32)
    i = jnp.int32(0x5F3759DF) - (i >> 1)
    y = lax.bitcast_convert_type(i, jnp.float32)
    for _ in range(3):
        y = y * (1.5 - 0.5 * d * y * y)
    return y


@functools.partial(
    pl.kernel,
    out_type=jax.ShapeDtypeStruct((N_PAD, HID), jnp.float32),
    mesh=_mesh,
    scratch_types=[
        pltpu.VMEM((ROWS_PER_WORKER, CHUNK), jnp.int32),
        pltpu.VMEM((CHUNK,), jnp.float32),
        pltpu.VMEM((SLICE_PER_TILE,), jnp.float32),
        pltpu.VMEM((DIS_PER_WORKER, HID), jnp.float32),
        pltpu.VMEM_SHARED((N_PAD,), jnp.float32),
    ],
    compiler_params=_sc_params,
)
def _deg_dis(dst_hbm, disb_hbm, dst_v, ones_v, buf_v, disb_v, deg_sh):
    c = lax.axis_index("c")
    s = lax.axis_index("s")
    for i in range(CHUNK // 16):
        ones_v[pl.ds(i * 16, 16)] = jnp.full((16,), 1.0, jnp.float32)
    for i in range(SLICE_PER_TILE // 16):
        buf_v[pl.ds(i * 16, 16)] = jnp.zeros((16,), jnp.float32)
    pltpu.sync_copy(buf_v, deg_sh.at[pl.ds(s * SLICE_PER_TILE, SLICE_PER_TILE)])
    plsc.subcore_barrier()
    for g in range(2):
        pltpu.sync_copy(dst_hbm.at[2 * s + g], dst_v)

        def body(j, carry):
            pltpu.sync_copy(ones_v, deg_sh.at[dst_v.at[j]], add=True)
            return carry

        lax.fori_loop(0, ROWS_PER_WORKER, body, 0)
    plsc.subcore_barrier()
    w = c * NS + s
    pltpu.sync_copy(deg_sh.at[pl.ds(w * DIS_PER_WORKER, DIS_PER_WORKER)],
                    buf_v.at[pl.ds(0, DIS_PER_WORKER)])
    for i in range(DIS_PER_WORKER // 16):
        d = buf_v[pl.ds(i * 16, 16)] + 1.0
        buf_v[pl.ds(i * 16, 16)] = _rsqrt16(d)

    def bbody(r, carry):
        dvec = buf_v[pl.ds((r // 16) * 16, 16)]
        sel = jnp.where(lax.iota(jnp.int32, 16) == (r % 16), dvec, 0.0)
        v = jnp.zeros((16,), jnp.float32) + jnp.sum(sel)
        for k in range(HID // 16):
            disb_v[r, pl.ds(k * 16, 16)] = v
        return carry

    lax.fori_loop(0, DIS_PER_WORKER, bbody, 0)
    pltpu.sync_copy(disb_v,
                    disb_hbm.at[pl.ds(w * DIS_PER_WORKER, DIS_PER_WORKER)])


@functools.partial(
    pl.kernel,
    out_type=jax.ShapeDtypeStruct((NC * N_TRIM, HID), jnp.float32),
    mesh=_mesh,
    scratch_types=[
        pltpu.VMEM((ROWS_PER_WORKER, CHUNK), jnp.int32),
        pltpu.VMEM((ROWS_PER_WORKER, CHUNK), jnp.int32),
        pltpu.VMEM((4, CHUNK, HID), jnp.float32),
        pltpu.VMEM_SHARED((N_PAD, HID), jnp.float32),
        pltpu.SemaphoreType.DMA,
        pltpu.SemaphoreType.DMA,
        pltpu.SemaphoreType.DMA,
        pltpu.SemaphoreType.DMA,
    ],
    compiler_params=_sc_params,
)
def _agg(src_hbm, dst_hbm, hs_hbm, out_hbm, src_v, dst_v, rows_v, agg_sh,
         sem0, sem1, sem2, sem3):
    c = lax.axis_index("c")
    s = lax.axis_index("s")
    w = c * NS + s

    def zbody(r, carry):
        for k in range(HID // 16):
            rows_v[0, r, pl.ds(k * 16, 16)] = jnp.zeros((16,), jnp.float32)
        return carry

    lax.fori_loop(0, CHUNK, zbody, 0)
    for i in range(SLICE_PER_TILE // CHUNK):
        pltpu.sync_copy(rows_v.at[0],
                        agg_sh.at[pl.ds(s * SLICE_PER_TILE + i * CHUNK, CHUNK)])
    plsc.subcore_barrier()

    pltpu.sync_copy(src_hbm.at[w], src_v)
    pltpu.sync_copy(dst_hbm.at[w], dst_v)

    sems = (sem0, sem1, sem2, sem3)

    def gather(j, b):
        pltpu.async_copy(hs_hbm.at[src_v.at[j]], rows_v.at[b], sems[b])

    for b in range(4):
        gather(b, b)

    n_t = ROWS_PER_WORKER // 4

    def body(t, carry):
        for b in range(4):
            j = 4 * t + b
            pltpu.make_async_copy(hs_hbm.at[src_v.at[j]], rows_v.at[b],
                                  sems[b]).wait()
            pltpu.sync_copy(rows_v.at[b], agg_sh.at[dst_v.at[j]], add=True)

            @pl.when(t < n_t - 1)
            def _():
                gather(j + 4, b)

        return carry

    lax.fori_loop(0, n_t, body, 0)
    plsc.subcore_barrier()
    base = s * SLICE_PER_TILE

    @pl.when(s < N_TRIM // SLICE_PER_TILE)
    def _():
        pltpu.sync_copy(
            agg_sh.at[pl.ds(base, SLICE_PER_TILE)],
            out_hbm.at[pl.ds(c * N_TRIM + base, SLICE_PER_TILE)])

    @pl.when(s == N_TRIM // SLICE_PER_TILE)
    def _():
        pltpu.sync_copy(
            agg_sh.at[pl.ds(base, N_TRIM % SLICE_PER_TILE)],
            out_hbm.at[pl.ds(c * N_TRIM + base, N_TRIM % SLICE_PER_TILE)])


def _mm_body(x_ref, w_ref, h_ref):
    h_ref[...] = jnp.dot(x_ref[...], w_ref[...],
                         preferred_element_type=jnp.float32)


_mm = pl.pallas_call(
    _mm_body,
    out_shape=jax.ShapeDtypeStruct((N_NODES, HID), jnp.float32),
)


def _scale_body(h_ref, disb_ref, hs_ref):
    hs_ref[...] = h_ref[...] * disb_ref[0:N_NODES, :]


_scale = pl.pallas_call(
    _scale_body,
    out_shape=jax.ShapeDtypeStruct((N_NODES, HID), jnp.float32),
)


def _combine_body(sp_ref, hs_ref, disb_ref, b_ref, act_ref):
    ssum = sp_ref[0:N_NODES, :] + sp_ref[N_TRIM:N_TRIM + N_NODES, :]
    a = (ssum + hs_ref[...]) * disb_ref[0:N_NODES, :] + b_ref[...]
    act_ref[...] = jnp.maximum(a, 0.0)


_combine = pl.pallas_call(
    _combine_body,
    out_shape=jax.ShapeDtypeStruct((N_NODES, HID), jnp.float32),
)


def _head_body(a_ref, w_ref, b_ref, o_ref):
    logits = jnp.dot(a_ref[...], w_ref[...],
                     preferred_element_type=jnp.float32) + b_ref[...]
    m = jnp.max(logits, axis=1, keepdims=True)
    lse = jnp.log(jnp.sum(jnp.exp(logits - m), axis=1, keepdims=True)) + m
    o_ref[...] = logits - lse


_head = pl.pallas_call(
    _head_body,
    out_shape=jax.ShapeDtypeStruct((N_GRAPHS, N_OUT), jnp.float32),
)


_N_EXTRA = E_PAD - N_EDGES
_PAD_SRC = np.arange(_N_EXTRA, dtype=np.int32) % N_NODES
_PAD_DST = (N_NODES
            + np.arange(_N_EXTRA, dtype=np.int32) % (N_PAD - N_NODES))


def kernel(x, edge_index, W_gcn, b_gcn, W_lin, b_lin):
    src3d = jnp.concatenate([edge_index[0], _PAD_SRC]).reshape(
        NW, ROWS_PER_WORKER, CHUNK)
    dst3d = jnp.concatenate([edge_index[1], _PAD_DST]).reshape(
        NW, ROWS_PER_WORKER, CHUNK)
    h = _mm(x, W_gcn)
    dis_b = _deg_dis(dst3d)
    hs = _scale(h, dis_b)
    s_part = _agg(src3d, dst3d, hs)
    act = _combine(s_part, hs, dis_b, b_gcn)
    act2 = act.reshape(N_GRAPHS, HID * NODES_PER_GRAPH)
    return _head(act2, W_lin, b_lin)
